# baseline (speedup 1.0000x reference)
import functools

import jax
import jax.numpy as jnp
from jax import lax
from jax.experimental import pallas as pl
from jax.experimental.pallas import tpu as pltpu
from jax.experimental.pallas import tpu_sc as plsc

NUM_ROWS = 16384
NUM_COLS = 200
EMBED_DIM = 64

NC = 2
NS = 16
NW = NC * NS

B = NUM_ROWS * NUM_COLS
I_PER_W = NUM_ROWS // NW
TI_PER_W = I_PER_W // 128
L = 16


def _make_kernel():
    mesh = plsc.VectorSubcoreMesh(core_axis_name="c", subcore_axis_name="s")

    @functools.partial(
        pl.kernel,
        mesh=mesh,
        out_type=jax.ShapeDtypeStruct((NUM_COLS, 8, 128, 8, 128),
                                      jnp.float32),
        scratch_types=[
            pltpu.VMEM((I_PER_W,), jnp.int32),
            pltpu.VMEM((I_PER_W,), jnp.int32),
            pltpu.VMEM((I_PER_W, EMBED_DIM), jnp.float32),
            pltpu.VMEM((I_PER_W, EMBED_DIM), jnp.float32),
            pltpu.VMEM((8, 8, 128), jnp.float32),
            pltpu.VMEM((8, 8, 128), jnp.float32),
            pltpu.VMEM((8, 8, 128), jnp.float32),
            pltpu.VMEM((8, 8, 128), jnp.float32),
            pltpu.SemaphoreType.DMA,
            pltpu.SemaphoreType.DMA,
            pltpu.SemaphoreType.DMA,
            pltpu.SemaphoreType.DMA,
            pltpu.SemaphoreType.DMA,
            pltpu.SemaphoreType.DMA,
            pltpu.SemaphoreType.DMA,
            pltpu.SemaphoreType.DMA,
        ],
        compiler_params=pltpu.CompilerParams(use_tc_tiling_on_sc=False,
                                             needs_layout_passes=False),
    )
    def gather_kernel(idxt_hbm, table_hbm, out_hbm,
                      idx0, idx1, a0, a1, bt0, bt1, bt2, bt3,
                      sg0, sg1, si0, si1, sb0, sb1, sb2, sb3):
        wid = lax.axis_index("s") * NC + lax.axis_index("c")
        i_base = wid * I_PER_W
        ti_base = wid * TI_PER_W
        idx_v = (idx0, idx1)
        a_v = (a0, a1)
        bt_v = (bt0, bt1, bt2, bt3)
        sg = (sg0, sg1)
        si = (si0, si1)
        sb = (sb0, sb1, sb2, sb3)

        iota = lax.iota(jnp.int32, L)

        def transpose_block(b, t, bt):
            a = a_v[b]
            dst = bt_v[bt]
            ivecs = [iota + (t * 128 + blk * L) for blk in range(8)]

            def tk_body(tk, carry):
                for kk in range(8):
                    kvec = jnp.full((L,), tk * 8 + kk, jnp.int32)
                    for blk in range(8):
                        v = plsc.load_gather(a, [ivecs[blk], kvec])
                        dst[tk, kk, pl.ds(blk * L, L)] = v
                return carry

            lax.fori_loop(0, 8, tk_body, 0)

        def store_block(j, t, bt, sem):
            return pltpu.async_copy(
                bt_v[bt], out_hbm.at[j, :, ti_base + t], sem)

        def wait_store_block(j, t, bt, sem):
            pltpu.make_async_copy(
                bt_v[bt], out_hbm.at[j, :, ti_base + t], sem).wait()

        pltpu.sync_copy(idxt_hbm.at[pl.ds(i_base, I_PER_W)], idx0)
        pltpu.async_copy(table_hbm.at[idx0], a0, sg0)
        pltpu.async_copy(idxt_hbm.at[pl.ds(NUM_ROWS + i_base, I_PER_W)],
                         idx1, si1)

        def pair(p, carry):
            for b in (0, 1):
                j = p * 2 + b
                nb = 1 - b

                pltpu.make_async_copy(table_hbm.at[idx_v[b]], a_v[b],
                                      sg[b]).wait()

                @pl.when(j + 1 < NUM_COLS)
                def _():
                    pltpu.make_async_copy(
                        idxt_hbm.at[pl.ds((j + 1) * NUM_ROWS + i_base,
                                          I_PER_W)],
                        idx_v[nb], si[nb]).wait()
                    pltpu.async_copy(table_hbm.at[idx_v[nb]], a_v[nb],
                                     sg[nb])

                @pl.when(j + 2 < NUM_COLS)
                def _():
                    pltpu.async_copy(
                        idxt_hbm.at[pl.ds((j + 2) * NUM_ROWS + i_base,
                                          I_PER_W)],
                        idx_v[b], si[b])

                for t in range(TI_PER_W):
                    @pl.when(j >= 1)
                    def _():
                        wait_store_block(j - 1, t, t, sb[t])

                    transpose_block(b, t, t)
                    store_block(j, t, t, sb[t])
            return carry

        lax.fori_loop(0, NUM_COLS // 2, pair, 0)

        for t in range(TI_PER_W):
            wait_store_block(NUM_COLS - 1, t, t, sb[t])

    return gather_kernel


_gather = _make_kernel()


@jax.jit
def kernel(x, table):
    idxt = x.T.reshape((B,)).astype(jnp.int32)
    out5 = _gather(idxt, table)
    return out5.transpose(2, 4, 0, 1, 3).reshape(
        (NUM_ROWS, NUM_COLS, EMBED_DIM))


# speedup vs baseline: 2.1294x; 2.1294x over previous
import functools

import jax
import jax.numpy as jnp
from jax import lax
from jax.experimental import pallas as pl
from jax.experimental.pallas import tpu as pltpu
from jax.experimental.pallas import tpu_sc as plsc

NUM_ROWS = 16384
NUM_COLS = 200
EMBED_DIM = 64
PAD_DIM = 65

NC = 2
NS = 16
NW = NC * NS

B = NUM_ROWS * NUM_COLS
I_PER_W = NUM_ROWS // NW
TI_PER_W = I_PER_W // 128
L = 16


def _make_kernel():
    mesh = plsc.VectorSubcoreMesh(core_axis_name="c", subcore_axis_name="s")

    @functools.partial(
        pl.kernel,
        mesh=mesh,
        out_type=jax.ShapeDtypeStruct((NUM_COLS, 8, 128, 8, 128),
                                      jnp.float32),
        scratch_types=[
            pltpu.VMEM((I_PER_W,), jnp.int32),
            pltpu.VMEM((I_PER_W,), jnp.int32),
            pltpu.VMEM((I_PER_W, PAD_DIM), jnp.float32),
            pltpu.VMEM((I_PER_W, PAD_DIM), jnp.float32),
            pltpu.VMEM((8, 8, 128), jnp.float32),
            pltpu.VMEM((8, 8, 128), jnp.float32),
            pltpu.VMEM((8, 8, 128), jnp.float32),
            pltpu.VMEM((8, 8, 128), jnp.float32),
            pltpu.SemaphoreType.DMA,
            pltpu.SemaphoreType.DMA,
            pltpu.SemaphoreType.DMA,
            pltpu.SemaphoreType.DMA,
            pltpu.SemaphoreType.DMA,
            pltpu.SemaphoreType.DMA,
            pltpu.SemaphoreType.DMA,
            pltpu.SemaphoreType.DMA,
        ],
        compiler_params=pltpu.CompilerParams(use_tc_tiling_on_sc=False,
                                             needs_layout_passes=False),
    )
    def gather_kernel(idxt_hbm, table_hbm, out_hbm,
                      idx0, idx1, a0, a1, bt0, bt1, bt2, bt3,
                      sg0, sg1, si0, si1, sb0, sb1, sb2, sb3):
        wid = lax.axis_index("s") * NC + lax.axis_index("c")
        i_base = wid * I_PER_W
        ti_base = wid * TI_PER_W
        idx_v = (idx0, idx1)
        a_v = (a0, a1)
        bt_v = (bt0, bt1, bt2, bt3)
        sg = (sg0, sg1)
        si = (si0, si1)
        sb = (sb0, sb1, sb2, sb3)

        iota = lax.iota(jnp.int32, L)

        def transpose_block(b, t, bt):
            a = a_v[b]
            dst = bt_v[bt]
            ivecs = [iota + (t * 128 + blk * L) for blk in range(8)]

            def tk_body(tk, carry):
                for kk in range(8):
                    kvec = jnp.full((L,), tk * 8 + kk, jnp.int32)
                    for blk in range(8):
                        v = plsc.load_gather(a, [ivecs[blk], kvec])
                        dst[tk, kk, pl.ds(blk * L, L)] = v
                return carry

            lax.fori_loop(0, 8, tk_body, 0)

        def store_block(j, t, bt, sem):
            return pltpu.async_copy(
                bt_v[bt], out_hbm.at[j, :, ti_base + t], sem)

        def wait_store_block(j, t, bt, sem):
            pltpu.make_async_copy(
                bt_v[bt], out_hbm.at[j, :, ti_base + t], sem).wait()

        pltpu.sync_copy(idxt_hbm.at[pl.ds(i_base, I_PER_W)], idx0)
        pltpu.async_copy(table_hbm.at[idx0], a0, sg0)
        pltpu.async_copy(idxt_hbm.at[pl.ds(NUM_ROWS + i_base, I_PER_W)],
                         idx1, si1)

        def pair(p, carry):
            for b in (0, 1):
                j = p * 2 + b
                nb = 1 - b

                pltpu.make_async_copy(table_hbm.at[idx_v[b]], a_v[b],
                                      sg[b]).wait()

                @pl.when(j + 1 < NUM_COLS)
                def _():
                    pltpu.make_async_copy(
                        idxt_hbm.at[pl.ds((j + 1) * NUM_ROWS + i_base,
                                          I_PER_W)],
                        idx_v[nb], si[nb]).wait()
                    pltpu.async_copy(table_hbm.at[idx_v[nb]], a_v[nb],
                                     sg[nb])

                @pl.when(j + 2 < NUM_COLS)
                def _():
                    pltpu.async_copy(
                        idxt_hbm.at[pl.ds((j + 2) * NUM_ROWS + i_base,
                                          I_PER_W)],
                        idx_v[b], si[b])

                for t in range(TI_PER_W):
                    @pl.when(j >= 1)
                    def _():
                        wait_store_block(j - 1, t, t, sb[t])

                    transpose_block(b, t, t)
                    store_block(j, t, t, sb[t])
            return carry

        lax.fori_loop(0, NUM_COLS // 2, pair, 0)

        for t in range(TI_PER_W):
            wait_store_block(NUM_COLS - 1, t, t, sb[t])

    return gather_kernel


_gather = _make_kernel()


@jax.jit
def kernel(x, table):
    idxt = x.T.reshape((B,)).astype(jnp.int32)
    table_pad = jnp.pad(table, ((0, 0), (0, PAD_DIM - EMBED_DIM)))
    out5 = _gather(idxt, table_pad)
    return out5.transpose(2, 4, 0, 1, 3).reshape(
        (NUM_ROWS, NUM_COLS, EMBED_DIM))


# speedup vs baseline: 2.8962x; 1.3601x over previous
import functools

import jax
import jax.numpy as jnp
from jax import lax
from jax.experimental import pallas as pl
from jax.experimental.pallas import tpu as pltpu
from jax.experimental.pallas import tpu_sc as plsc

NUM_ROWS = 16384
NUM_COLS = 200
EMBED_DIM = 64

NC = 2
NS = 16
NW = NC * NS

B = NUM_ROWS * NUM_COLS
I_PER_W = NUM_ROWS // NW
TI_PER_W = I_PER_W // 128
L = 16


def _make_kernel():
    mesh = plsc.VectorSubcoreMesh(core_axis_name="c", subcore_axis_name="s")

    @functools.partial(
        pl.kernel,
        mesh=mesh,
        out_type=jax.ShapeDtypeStruct((NUM_COLS, 8, 128, 8, 128),
                                      jnp.float32),
        scratch_types=[
            pltpu.VMEM((I_PER_W,), jnp.int32),
            pltpu.VMEM((I_PER_W,), jnp.int32),
            pltpu.VMEM((I_PER_W, EMBED_DIM), jnp.float32),
            pltpu.VMEM((I_PER_W, EMBED_DIM), jnp.float32),
            pltpu.VMEM((8, 8, 128), jnp.float32),
            pltpu.VMEM((8, 8, 128), jnp.float32),
            pltpu.VMEM((8, 8, 128), jnp.float32),
            pltpu.VMEM((8, 8, 128), jnp.float32),
            pltpu.SemaphoreType.DMA,
            pltpu.SemaphoreType.DMA,
            pltpu.SemaphoreType.DMA,
            pltpu.SemaphoreType.DMA,
            pltpu.SemaphoreType.DMA,
            pltpu.SemaphoreType.DMA,
            pltpu.SemaphoreType.DMA,
            pltpu.SemaphoreType.DMA,
        ],
        compiler_params=pltpu.CompilerParams(use_tc_tiling_on_sc=False,
                                             needs_layout_passes=False),
    )
    def gather_kernel(idxt_hbm, table_hbm, out_hbm,
                      idx0, idx1, a0, a1, bt0, bt1, bt2, bt3,
                      sg0, sg1, si0, si1, sb0, sb1, sb2, sb3):
        wid = lax.axis_index("s") * NC + lax.axis_index("c")
        i_base = wid * I_PER_W
        ti_base = wid * TI_PER_W
        idx_v = (idx0, idx1)
        a_v = (a0, a1)
        bt_v = (bt0, bt1, bt2, bt3)
        sg = (sg0, sg1)
        si = (si0, si1)
        sb = (sb0, sb1, sb2, sb3)

        iota = lax.iota(jnp.int32, L)

        def transpose_block(b, t, bt):
            a = a_v[b]
            dst = bt_v[bt]
            ivecs = [iota + (t * 128 + blk * L) for blk in range(8)]
            iivecs = [iota + blk * L for blk in range(8)]

            def k_body(k, carry):
                kpl = jnp.bitwise_and(k + iota, 63)
                tkv = jnp.right_shift(kpl, 3)
                kkv = jnp.bitwise_and(kpl, 7)
                for blk in range(8):
                    v = plsc.load_gather(a, [ivecs[blk], kpl])
                    plsc.store_scatter(dst, [tkv, kkv, iivecs[blk]], v)
                return carry

            lax.fori_loop(0, EMBED_DIM, k_body, 0)

        def store_block(j, t, bt, sem):
            return pltpu.async_copy(
                bt_v[bt], out_hbm.at[j, :, ti_base + t], sem)

        def wait_store_block(j, t, bt, sem):
            pltpu.make_async_copy(
                bt_v[bt], out_hbm.at[j, :, ti_base + t], sem).wait()

        pltpu.sync_copy(idxt_hbm.at[pl.ds(i_base, I_PER_W)], idx0)
        pltpu.async_copy(table_hbm.at[idx0], a0, sg0)
        pltpu.async_copy(idxt_hbm.at[pl.ds(NUM_ROWS + i_base, I_PER_W)],
                         idx1, si1)

        def pair(p, carry):
            for b in (0, 1):
                j = p * 2 + b
                nb = 1 - b

                pltpu.make_async_copy(table_hbm.at[idx_v[b]], a_v[b],
                                      sg[b]).wait()

                @pl.when(j + 1 < NUM_COLS)
                def _():
                    pltpu.make_async_copy(
                        idxt_hbm.at[pl.ds((j + 1) * NUM_ROWS + i_base,
                                          I_PER_W)],
                        idx_v[nb], si[nb]).wait()
                    pltpu.async_copy(table_hbm.at[idx_v[nb]], a_v[nb],
                                     sg[nb])

                @pl.when(j + 2 < NUM_COLS)
                def _():
                    pltpu.async_copy(
                        idxt_hbm.at[pl.ds((j + 2) * NUM_ROWS + i_base,
                                          I_PER_W)],
                        idx_v[b], si[b])

                for t in range(TI_PER_W):
                    @pl.when(j >= 1)
                    def _():
                        wait_store_block(j - 1, t, t, sb[t])

                    transpose_block(b, t, t)
                    store_block(j, t, t, sb[t])
            return carry

        lax.fori_loop(0, NUM_COLS // 2, pair, 0)

        for t in range(TI_PER_W):
            wait_store_block(NUM_COLS - 1, t, t, sb[t])

    return gather_kernel


_gather = _make_kernel()


@jax.jit
def kernel(x, table):
    idxt = x.T.reshape((B,)).astype(jnp.int32)
    out5 = _gather(idxt, table)
    return out5.transpose(2, 4, 0, 1, 3).reshape(
        (NUM_ROWS, NUM_COLS, EMBED_DIM))


# speedup vs baseline: 2.8972x; 1.0003x over previous
import functools

import jax
import jax.numpy as jnp
from jax import lax
from jax.experimental import pallas as pl
from jax.experimental.pallas import tpu as pltpu
from jax.experimental.pallas import tpu_sc as plsc

NUM_ROWS = 16384
NUM_COLS = 200
EMBED_DIM = 64

NC = 2
NS = 16
NW = NC * NS

B = NUM_ROWS * NUM_COLS
I_PER_W = NUM_ROWS // NW
TI_PER_W = I_PER_W // 128
L = 16


def _make_kernel():
    mesh = plsc.VectorSubcoreMesh(core_axis_name="c", subcore_axis_name="s")

    @functools.partial(
        pl.kernel,
        mesh=mesh,
        out_type=jax.ShapeDtypeStruct((NUM_COLS, 8, 128, 1024),
                                      jnp.float32),
        scratch_types=[
            pltpu.VMEM((I_PER_W,), jnp.int32),
            pltpu.VMEM((I_PER_W,), jnp.int32),
            pltpu.VMEM((I_PER_W, EMBED_DIM), jnp.float32),
            pltpu.VMEM((I_PER_W, EMBED_DIM), jnp.float32),
            pltpu.VMEM((8, 1024), jnp.float32),
            pltpu.VMEM((8, 1024), jnp.float32),
            pltpu.VMEM((8, 1024), jnp.float32),
            pltpu.VMEM((8, 1024), jnp.float32),
            pltpu.SemaphoreType.DMA,
            pltpu.SemaphoreType.DMA,
            pltpu.SemaphoreType.DMA,
            pltpu.SemaphoreType.DMA,
            pltpu.SemaphoreType.DMA,
            pltpu.SemaphoreType.DMA,
            pltpu.SemaphoreType.DMA,
            pltpu.SemaphoreType.DMA,
        ],
        compiler_params=pltpu.CompilerParams(use_tc_tiling_on_sc=False,
                                             needs_layout_passes=False),
    )
    def gather_kernel(idxt_hbm, table_hbm, out_hbm,
                      idx0, idx1, a0, a1, bt0, bt1, bt2, bt3,
                      sg0, sg1, si0, si1, sb0, sb1, sb2, sb3):
        wid = lax.axis_index("s") * NC + lax.axis_index("c")
        i_base = wid * I_PER_W
        ti_base = wid * TI_PER_W
        idx_v = (idx0, idx1)
        a_v = (a0, a1)
        bt_v = (bt0, bt1, bt2, bt3)
        sg = (sg0, sg1)
        si = (si0, si1)
        sb = (sb0, sb1, sb2, sb3)

        iota = lax.iota(jnp.int32, L)

        def transpose_block(b, t, bt):
            a = a_v[b]
            dst = bt_v[bt]
            ivecs = [iota + (t * 128 + blk * L) for blk in range(8)]
            iivecs = [iota + blk * L for blk in range(8)]

            def k_body(k, carry):
                kpl = jnp.bitwise_and(k + iota, 63)
                tkv = jnp.right_shift(kpl, 3)
                kk128 = jnp.left_shift(jnp.bitwise_and(kpl, 7), 7)
                for blk in range(8):
                    v = plsc.load_gather(a, [ivecs[blk], kpl])
                    plsc.store_scatter(dst, [tkv, kk128 + iivecs[blk]], v)
                return carry

            lax.fori_loop(0, EMBED_DIM, k_body, 0)

        def store_block(j, t, bt, sem):
            return pltpu.async_copy(
                bt_v[bt], out_hbm.at[j, :, ti_base + t], sem)

        def wait_store_block(j, t, bt, sem):
            pltpu.make_async_copy(
                bt_v[bt], out_hbm.at[j, :, ti_base + t], sem).wait()

        pltpu.sync_copy(idxt_hbm.at[pl.ds(i_base, I_PER_W)], idx0)
        pltpu.async_copy(table_hbm.at[idx0], a0, sg0)
        pltpu.async_copy(idxt_hbm.at[pl.ds(NUM_ROWS + i_base, I_PER_W)],
                         idx1, si1)

        def pair(p, carry):
            for b in (0, 1):
                j = p * 2 + b
                nb = 1 - b

                pltpu.make_async_copy(table_hbm.at[idx_v[b]], a_v[b],
                                      sg[b]).wait()

                @pl.when(j + 1 < NUM_COLS)
                def _():
                    pltpu.make_async_copy(
                        idxt_hbm.at[pl.ds((j + 1) * NUM_ROWS + i_base,
                                          I_PER_W)],
                        idx_v[nb], si[nb]).wait()
                    pltpu.async_copy(table_hbm.at[idx_v[nb]], a_v[nb],
                                     sg[nb])

                @pl.when(j + 2 < NUM_COLS)
                def _():
                    pltpu.async_copy(
                        idxt_hbm.at[pl.ds((j + 2) * NUM_ROWS + i_base,
                                          I_PER_W)],
                        idx_v[b], si[b])

                for t in range(TI_PER_W):
                    @pl.when(j >= 1)
                    def _():
                        wait_store_block(j - 1, t, t, sb[t])

                    transpose_block(b, t, t)
                    store_block(j, t, t, sb[t])
            return carry

        lax.fori_loop(0, NUM_COLS // 2, pair, 0)

        for t in range(TI_PER_W):
            wait_store_block(NUM_COLS - 1, t, t, sb[t])

    return gather_kernel


_gather = _make_kernel()


@jax.jit
def kernel(x, table):
    idxt = x.T.reshape((B,)).astype(jnp.int32)
    out4 = _gather(idxt, table)
    out5 = out4.reshape((NUM_COLS, 8, 128, 8, 128))
    return out5.transpose(2, 4, 0, 1, 3).reshape(
        (NUM_ROWS, NUM_COLS, EMBED_DIM))
